# baseline (speedup 1.0000x reference)
import jax
import jax.numpy as jnp
from jax import lax
from jax.experimental import pallas as pl
from jax.experimental.pallas import tpu as pltpu
from jax.experimental.pallas import tpu_sc as plsc

_N = 100000
_D = 768
_A = 64
_H = 256
_K = 16
_BLK = 2048
_GRID = 49
_NPAD = _GRID * _BLK
_NTILE = 16
_CHUNK = _NPAD // _NTILE
_NCH = _CHUNK // 16


def _scores_body(x_ref, w1_ref, b1_ref, w2_ref, out_ref):
    i = pl.program_id(0)
    xb = x_ref[...].astype(jnp.bfloat16)
    w1 = w1_ref[...].astype(jnp.bfloat16)
    h = jnp.tanh(
        lax.dot_general(xb, w1, (((1,), (0,)), ((), ())),
                        preferred_element_type=jnp.float32)
        + b1_ref[...]
    )
    hb = h.astype(jnp.bfloat16).astype(jnp.float32)
    w2 = w2_ref[...].astype(jnp.bfloat16).astype(jnp.float32)
    s = jnp.sum(hb * w2, axis=1)
    row = i * _BLK + lax.broadcasted_iota(jnp.int32, (_BLK,), 0)
    out_ref[0, 0, :] = jnp.where(row < _N, s, -jnp.inf)


def _scores(x, w1, b1_2d, w2_2d):
    return pl.pallas_call(
        _scores_body,
        grid=(_GRID,),
        in_specs=[
            pl.BlockSpec((_BLK, _D), lambda i: (i, 0)),
            pl.BlockSpec((_D, _A), lambda i: (0, 0)),
            pl.BlockSpec((1, _A), lambda i: (0, 0)),
            pl.BlockSpec((1, _A), lambda i: (0, 0)),
        ],
        out_specs=pl.BlockSpec((1, 1, _BLK), lambda i: (i, 0, 0)),
        out_shape=jax.ShapeDtypeStruct((_GRID, 1, _BLK), jnp.float32),
    )(x, w1, b1_2d, w2_2d)


def _merge16(rv, ri, cv, ci):
    cv, ci = plsc.sort_key_val(cv, ci, descending=True)
    take = cv > rv
    nv = jnp.where(take, cv, rv)
    ni = jnp.where(take, ci, ri)
    nv, ni = plsc.sort_key_val(nv, ni, descending=False)
    return nv, ni


def _mil_sc_body(scores_hbm, x_hbm, wc1t_hbm, bc1_hbm, wc2_hbm, bc2_hbm,
                 fw_hbm, idx_hbm, emb_hbm, lg_hbm, exv_hbm, exi_hbm, exh_hbm,
                 sc_v, zb_v, rows_v, wrow_v, emb_vv, hall_v,
                 allv_v, alli_v, stgf, stgi, stge, wc2_v, bc1_v, bc2_v, sem):
    sid = lax.axis_index("s")
    base = sid * _CHUNK
    lane = lax.iota(jnp.int32, 16)
    ninf = jnp.full((16,), -jnp.inf, jnp.float32)
    zi = jnp.zeros((16,), jnp.int32)

    pltpu.sync_copy(scores_hbm.at[pl.ds(base, _CHUNK)], sc_v)

    def scan_body(c, carry):
        rv, ri, tmin = carry
        cv = sc_v[pl.ds(c * 16, 16)]

        def do_merge(args):
            mv, mi = args
            ci = base + c * 16 + lane
            nv, ni = _merge16(mv, mi, cv, ci)
            return nv, ni, jnp.min(nv)

        return lax.cond(jnp.any(cv > tmin), do_merge,
                        lambda args: (args[0], args[1], tmin), (rv, ri))

    rv, ri, _ = lax.fori_loop(
        0, _NCH, scan_body, (ninf, zi, -jnp.inf), unroll=False)

    stgf[...] = rv
    stgi[...] = ri
    pltpu.sync_copy(stgf, exv_hbm.at[sid])
    pltpu.sync_copy(stgi, exi_hbm.at[sid])
    plsc.subcore_barrier()

    pltpu.sync_copy(exv_hbm, allv_v)
    pltpu.sync_copy(exi_hbm, alli_v)
    rv, ri = ninf, zi
    for t in range(_NTILE):
        rv, ri = _merge16(rv, ri, allv_v[t, :], alli_v[t, :])
    top_v = lax.rev(rv, (0,))
    top_i = lax.rev(ri, (0,))

    mx = jnp.max(top_v)
    e = jnp.exp(top_v - mx)
    w = e / jnp.sum(e)

    stgi[...] = top_i

    @pl.when(sid == 0)
    def _():
        pltpu.sync_copy(stgi, idx_hbm)

    def zb_body(c, carry):
        zb_v[pl.ds(c * 16, 16)] = jnp.zeros((16,), jnp.float32)
        return carry

    lax.fori_loop(0, _NCH, zb_body, 0)
    li = top_i - base
    msk = (li >= 0) & (li < _CHUNK)
    plsc.store_scatter(zb_v, [jnp.where(msk, li, 0)], w, mask=msk)
    pltpu.sync_copy(zb_v, fw_hbm.at[pl.ds(base, _CHUNK)])

    pltpu.async_copy(x_hbm.at[stgi], rows_v, sem).wait()

    wk = []
    for k in range(_K):
        wk.append(jnp.sum(jnp.where(lane == k, w, 0.0)))
    for jj in range(3):
        j = sid * 3 + jj
        acc = jnp.zeros((16,), jnp.float32)
        for k in range(_K):
            acc = acc + wk[k] * rows_v[k, pl.ds(j * 16, 16)]
        stge[pl.ds(jj * 16, 16)] = acc
    pltpu.sync_copy(stge, emb_hbm.at[pl.ds(sid * 48, 48)])
    plsc.subcore_barrier()

    pltpu.sync_copy(emb_hbm, emb_vv)
    pltpu.sync_copy(wc1t_hbm.at[pl.ds(sid * 16, 16), :], wrow_v)
    pltpu.sync_copy(bc1_hbm.at[pl.ds(sid * 16, 16)], bc1_v)
    hvec = jnp.zeros((16,), jnp.float32)
    for cc in range(16):
        def hb(jj, acc, _cc=cc):
            return acc + emb_vv[pl.ds(jj * 16, 16)] * wrow_v[_cc, pl.ds(jj * 16, 16)]
        acc = lax.fori_loop(0, _D // 16, hb, jnp.zeros((16,), jnp.float32))
        hvec = hvec + jnp.where(lane == cc, jnp.sum(acc), 0.0)
    hvec = jnp.maximum(hvec + bc1_v[...], 0.0)
    stgf[...] = hvec
    pltpu.sync_copy(stgf, exh_hbm.at[sid])
    plsc.subcore_barrier()

    @pl.when(sid == 0)
    def _():
        pltpu.sync_copy(exh_hbm, hall_v)
        pltpu.sync_copy(wc2_hbm, wc2_v)
        pltpu.sync_copy(bc2_hbm, bc2_v)
        acc = jnp.zeros((16,), jnp.float32)
        for jj in range(16):
            acc = acc + hall_v[jj, :] * wc2_v[pl.ds(jj * 16, 16)]
        stgf[...] = jnp.full((16,), jnp.sum(acc), jnp.float32) + bc2_v[...]
        pltpu.sync_copy(stgf, lg_hbm)


def _mil_sc(scores, x, wc1t, bc1, wc2, bc2v):
    mesh = plsc.VectorSubcoreMesh(core_axis_name="c", subcore_axis_name="s")
    fn = pl.kernel(
        _mil_sc_body,
        out_type=(
            jax.ShapeDtypeStruct((_NPAD,), jnp.float32),
            jax.ShapeDtypeStruct((_K,), jnp.int32),
            jax.ShapeDtypeStruct((_D,), jnp.float32),
            jax.ShapeDtypeStruct((16,), jnp.float32),
            jax.ShapeDtypeStruct((_NTILE, 16), jnp.float32),
            jax.ShapeDtypeStruct((_NTILE, 16), jnp.int32),
            jax.ShapeDtypeStruct((_NTILE, 16), jnp.float32),
        ),
        mesh=mesh,
        compiler_params=pltpu.CompilerParams(needs_layout_passes=False),
        scratch_types=[
            pltpu.VMEM((_CHUNK,), jnp.float32),
            pltpu.VMEM((_CHUNK,), jnp.float32),
            pltpu.VMEM((_K, _D), jnp.float32),
            pltpu.VMEM((16, _D), jnp.float32),
            pltpu.VMEM((_D,), jnp.float32),
            pltpu.VMEM((16, 16), jnp.float32),
            pltpu.VMEM((16, 16), jnp.float32),
            pltpu.VMEM((16, 16), jnp.int32),
            pltpu.VMEM((16,), jnp.float32),
            pltpu.VMEM((16,), jnp.int32),
            pltpu.VMEM((48,), jnp.float32),
            pltpu.VMEM((_H,), jnp.float32),
            pltpu.VMEM((16,), jnp.float32),
            pltpu.VMEM((16,), jnp.float32),
            pltpu.SemaphoreType.DMA,
        ],
    )
    return fn(scores, x, wc1t, bc1, wc2, bc2v)


def kernel(x, W1, b1, W2, b2, Wc1, bc1, Wc2, bc2):
    del b2
    scores = _scores(x, W1, b1.reshape(1, _A), W2.reshape(1, _A)).reshape(_NPAD)
    wc1t = Wc1.T
    fw_pad, topk_idx, emb, lg = _mil_sc(
        scores, x, wc1t, bc1, Wc2.reshape(_H), jnp.broadcast_to(bc2, (16,))
    )[:4]
    return (lg[0], emb, fw_pad[:_N], topk_idx)


# speedup vs baseline: 1.1422x; 1.1422x over previous
import jax
import jax.numpy as jnp
from jax import lax
from jax.experimental import pallas as pl
from jax.experimental.pallas import tpu as pltpu
from jax.experimental.pallas import tpu_sc as plsc

_N = 100000
_D = 768
_A = 64
_H = 256
_K = 16
_BLK = 4096
_GRID = 25
_NPAD = _GRID * _BLK
_NTILE = 16
_CHUNK = _NPAD // _NTILE
_NCH = _CHUNK // 16


def _scores_body(x_ref, w1_ref, b1_ref, w2_ref, out_ref):
    i = pl.program_id(0)
    xb = x_ref[...].astype(jnp.bfloat16)
    w1 = w1_ref[...].astype(jnp.bfloat16)
    h = jnp.tanh(
        lax.dot_general(xb, w1, (((1,), (0,)), ((), ())),
                        preferred_element_type=jnp.float32)
        + b1_ref[...]
    )
    hb = h.astype(jnp.bfloat16).astype(jnp.float32)
    w2 = w2_ref[...].astype(jnp.bfloat16).astype(jnp.float32)
    s = jnp.sum(hb * w2, axis=1)
    row = i * _BLK + lax.broadcasted_iota(jnp.int32, (_BLK,), 0)
    out_ref[0, 0, :] = jnp.where(row < _N, s, -jnp.inf)


def _scores(x, w1, b1_2d, w2_2d):
    return pl.pallas_call(
        _scores_body,
        grid=(_GRID,),
        in_specs=[
            pl.BlockSpec((_BLK, _D), lambda i: (i, 0)),
            pl.BlockSpec((_D, _A), lambda i: (0, 0)),
            pl.BlockSpec((1, _A), lambda i: (0, 0)),
            pl.BlockSpec((1, _A), lambda i: (0, 0)),
        ],
        out_specs=pl.BlockSpec((1, 1, _BLK), lambda i: (i, 0, 0)),
        out_shape=jax.ShapeDtypeStruct((_GRID, 1, _BLK), jnp.float32),
    )(x, w1, b1_2d, w2_2d)


def _merge16(rv, ri, cv, ci):
    cv, ci = plsc.sort_key_val(cv, ci, descending=True)
    take = cv > rv
    nv = jnp.where(take, cv, rv)
    ni = jnp.where(take, ci, ri)
    nv, ni = plsc.sort_key_val(nv, ni, descending=False)
    return nv, ni


def _mil_sc_body(scores_hbm, x_hbm, wc1t_hbm, bc1_hbm, wc2_hbm, bc2_hbm,
                 fw_hbm, idx_hbm, emb_hbm, lg_hbm, exv_hbm, exi_hbm, exh_hbm,
                 sc_v, zb_v, rows_v, wrow_v, emb_vv, hall_v,
                 allv_v, alli_v, stgf, stgi, stge, wc2_v, bc1_v, bc2_v, sem):
    sid = lax.axis_index("s")
    base = sid * _CHUNK
    lane = lax.iota(jnp.int32, 16)
    ninf = jnp.full((16,), -jnp.inf, jnp.float32)
    zi = jnp.zeros((16,), jnp.int32)

    pltpu.sync_copy(scores_hbm.at[pl.ds(base, _CHUNK)], sc_v)

    def scan_body(c, carry):
        rv, ri = carry
        cv = sc_v[pl.ds(c * 16, 16)]
        ci = base + c * 16 + lane
        return _merge16(rv, ri, cv, ci)

    rv, ri = lax.fori_loop(0, _NCH, scan_body, (ninf, zi))

    stgf[...] = rv
    stgi[...] = ri
    pltpu.sync_copy(stgf, exv_hbm.at[sid])
    pltpu.sync_copy(stgi, exi_hbm.at[sid])
    plsc.subcore_barrier()

    pltpu.sync_copy(exv_hbm, allv_v)
    pltpu.sync_copy(exi_hbm, alli_v)
    rv, ri = ninf, zi
    for t in range(_NTILE):
        rv, ri = _merge16(rv, ri, allv_v[t, :], alli_v[t, :])
    top_v = lax.rev(rv, (0,))
    top_i = lax.rev(ri, (0,))

    mx = jnp.max(top_v)
    e = jnp.exp(top_v - mx)
    w = e / jnp.sum(e)

    stgi[...] = top_i

    @pl.when(sid == 0)
    def _():
        pltpu.sync_copy(stgi, idx_hbm)

    def zb_body(c, carry):
        zb_v[pl.ds(c * 16, 16)] = jnp.zeros((16,), jnp.float32)
        return carry

    lax.fori_loop(0, _NCH, zb_body, 0)
    li = top_i - base
    msk = (li >= 0) & (li < _CHUNK)
    plsc.store_scatter(zb_v, [jnp.where(msk, li, 0)], w, mask=msk)
    pltpu.sync_copy(zb_v, fw_hbm.at[pl.ds(base, _CHUNK)])

    pltpu.async_copy(x_hbm.at[stgi], rows_v, sem).wait()

    wk = []
    for k in range(_K):
        wk.append(jnp.sum(jnp.where(lane == k, w, 0.0)))
    for jj in range(3):
        j = sid * 3 + jj
        acc = jnp.zeros((16,), jnp.float32)
        for k in range(_K):
            acc = acc + wk[k] * rows_v[k, pl.ds(j * 16, 16)]
        stge[pl.ds(jj * 16, 16)] = acc
    pltpu.sync_copy(stge, emb_hbm.at[pl.ds(sid * 48, 48)])
    plsc.subcore_barrier()

    pltpu.sync_copy(emb_hbm, emb_vv)
    pltpu.sync_copy(wc1t_hbm.at[pl.ds(sid * 16, 16), :], wrow_v)
    pltpu.sync_copy(bc1_hbm.at[pl.ds(sid * 16, 16)], bc1_v)
    hvec = jnp.zeros((16,), jnp.float32)
    for cc in range(16):
        def hb(jj, acc, _cc=cc):
            return acc + emb_vv[pl.ds(jj * 16, 16)] * wrow_v[_cc, pl.ds(jj * 16, 16)]
        acc = lax.fori_loop(0, _D // 16, hb, jnp.zeros((16,), jnp.float32))
        hvec = hvec + jnp.where(lane == cc, jnp.sum(acc), 0.0)
    hvec = jnp.maximum(hvec + bc1_v[...], 0.0)
    stgf[...] = hvec
    pltpu.sync_copy(stgf, exh_hbm.at[sid])
    plsc.subcore_barrier()

    @pl.when(sid == 0)
    def _():
        pltpu.sync_copy(exh_hbm, hall_v)
        pltpu.sync_copy(wc2_hbm, wc2_v)
        pltpu.sync_copy(bc2_hbm, bc2_v)
        acc = jnp.zeros((16,), jnp.float32)
        for jj in range(16):
            acc = acc + hall_v[jj, :] * wc2_v[pl.ds(jj * 16, 16)]
        stgf[...] = jnp.full((16,), jnp.sum(acc), jnp.float32) + bc2_v[...]
        pltpu.sync_copy(stgf, lg_hbm)


def _mil_sc(scores, x, wc1t, bc1, wc2, bc2v):
    mesh = plsc.VectorSubcoreMesh(core_axis_name="c", subcore_axis_name="s")
    fn = pl.kernel(
        _mil_sc_body,
        out_type=(
            jax.ShapeDtypeStruct((_NPAD,), jnp.float32),
            jax.ShapeDtypeStruct((_K,), jnp.int32),
            jax.ShapeDtypeStruct((_D,), jnp.float32),
            jax.ShapeDtypeStruct((16,), jnp.float32),
            jax.ShapeDtypeStruct((_NTILE, 16), jnp.float32),
            jax.ShapeDtypeStruct((_NTILE, 16), jnp.int32),
            jax.ShapeDtypeStruct((_NTILE, 16), jnp.float32),
        ),
        mesh=mesh,
        compiler_params=pltpu.CompilerParams(needs_layout_passes=False),
        scratch_types=[
            pltpu.VMEM((_CHUNK,), jnp.float32),
            pltpu.VMEM((_CHUNK,), jnp.float32),
            pltpu.VMEM((_K, _D), jnp.float32),
            pltpu.VMEM((16, _D), jnp.float32),
            pltpu.VMEM((_D,), jnp.float32),
            pltpu.VMEM((16, 16), jnp.float32),
            pltpu.VMEM((16, 16), jnp.float32),
            pltpu.VMEM((16, 16), jnp.int32),
            pltpu.VMEM((16,), jnp.float32),
            pltpu.VMEM((16,), jnp.int32),
            pltpu.VMEM((48,), jnp.float32),
            pltpu.VMEM((_H,), jnp.float32),
            pltpu.VMEM((16,), jnp.float32),
            pltpu.VMEM((16,), jnp.float32),
            pltpu.SemaphoreType.DMA,
        ],
    )
    return fn(scores, x, wc1t, bc1, wc2, bc2v)


def kernel(x, W1, b1, W2, b2, Wc1, bc1, Wc2, bc2):
    del b2
    scores = _scores(x, W1, b1.reshape(1, _A), W2.reshape(1, _A)).reshape(_NPAD)
    wc1t = Wc1.T
    fw_pad, topk_idx, emb, lg = _mil_sc(
        scores, x, wc1t, bc1, Wc2.reshape(_H), jnp.broadcast_to(bc2, (16,))
    )[:4]
    return (lg[0], emb, fw_pad[:_N], topk_idx)
